# baseline (speedup 1.0000x reference)
import jax
import jax.numpy as jnp
from jax.experimental import pallas as pl

_CHUNK = 1024


def _add_kernel(x_ref, pos_ref, out_ref):
    out_ref[...] = x_ref[...] + pos_ref[...][None, :, :]


def kernel(x, pos_table):
    batch, max_len, dim = x.shape
    grid = (max_len // _CHUNK, batch)
    return pl.pallas_call(
        _add_kernel,
        grid=grid,
        in_specs=[
            pl.BlockSpec((1, _CHUNK, dim), lambda i, b: (b, i, 0)),
            pl.BlockSpec((_CHUNK, dim), lambda i, b: (i, 0)),
        ],
        out_specs=pl.BlockSpec((1, _CHUNK, dim), lambda i, b: (b, i, 0)),
        out_shape=jax.ShapeDtypeStruct(x.shape, x.dtype),
    )(x, pos_table)


# speedup vs baseline: 1.0664x; 1.0664x over previous
import jax
import jax.numpy as jnp
from jax.experimental import pallas as pl

_CHUNK = 1024


def _add_kernel(x_ref, pos_ref, out_ref):
    out_ref[...] = x_ref[...] + pos_ref[...][None, :, :]


def kernel(x, pos_table):
    batch, max_len, dim = x.shape
    grid = (max_len // _CHUNK,)
    return pl.pallas_call(
        _add_kernel,
        grid=grid,
        in_specs=[
            pl.BlockSpec((batch, _CHUNK, dim), lambda i: (0, i, 0)),
            pl.BlockSpec((_CHUNK, dim), lambda i: (i, 0)),
        ],
        out_specs=pl.BlockSpec((batch, _CHUNK, dim), lambda i: (0, i, 0)),
        out_shape=jax.ShapeDtypeStruct(x.shape, x.dtype),
    )(x, pos_table)
